# Initial kernel scaffold; baseline (speedup 1.0000x reference)
import jax
import jax.numpy as jnp
from jax.experimental import pallas as pl


def kernel(x, cnn_w1, cnn_b1, cnn_w2, cnn_b2, cnn_w3, cnn_b3, fc_w, fc_b, norm_w, in_proj_w, conv1d_w, conv1d_b, x_proj_w, dt_proj_w, dt_proj_b, A_log, Dp, out_proj_w, norm_f_w, head_w1, head_b1, head_w2, head_b2):
    raise NotImplementedError("write your pallas kernel here")


# speedup vs baseline: 4.3302x; 4.3302x over previous
import jax
import jax.numpy as jnp
from jax.experimental import pallas as pl
from jax.experimental.pallas import tpu as pltpu

_B, _T, _H, _W = 8, 256, 64, 64
_D = 256
_E = 512
_N = 16
_DTR = 16
_NL = 6
_FB = 64
_F = _B * _T

_TAPS = [(di, dj) for di in range(3) for dj in range(3)]


def _cnn_kernel(p1_ref, w1_ref, b1_ref, w2_ref, b2_ref, w3_ref, b3_ref,
                fcw_ref, fcb_ref, o_ref, ap1_ref, ap2a_ref, ap2b_ref):
    p1 = p1_ref[...].reshape(8 * 32 * 32, 72)
    a1 = jnp.maximum(jnp.dot(p1, w1_ref[...],
                             preferred_element_type=jnp.float32)
                     + b1_ref[...], 0.0)
    ap1_ref[:, 0:32, 0:32, :] = a1.reshape(8, 32, 32, 128)
    ap1_ref[:, 32:34, :, :] = jnp.zeros((8, 2, 34, 128), jnp.float32)
    ap1_ref[:, 0:32, 32:34, :] = jnp.zeros((8, 32, 2, 128), jnp.float32)

    acc = jnp.zeros((8 * 16 * 16, 256), jnp.float32)
    for s, (di, dj) in enumerate(_TAPS):
        v = ap1_ref[:, pl.ds(di, 16, 2), pl.ds(dj, 16, 2), :]
        acc = acc + jnp.dot(v.reshape(8 * 16 * 16, 128), w2_ref[s],
                            preferred_element_type=jnp.float32)
    a2 = jnp.maximum(acc + b2_ref[...], 0.0)
    a2 = a2.reshape(8, 16, 16, 256)
    for r, lo in ((ap2a_ref, 0), (ap2b_ref, 128)):
        r[:, 0:16, 0:16, :] = a2[..., lo:lo + 128]
        r[:, 16:18, :, :] = jnp.zeros((8, 2, 18, 128), jnp.float32)
        r[:, 0:16, 16:18, :] = jnp.zeros((8, 16, 2, 128), jnp.float32)

    acc = jnp.zeros((8 * 8 * 8, 256), jnp.float32)
    for s, (di, dj) in enumerate(_TAPS):
        v = jnp.concatenate(
            [ap2a_ref[:, pl.ds(di, 8, 2), pl.ds(dj, 8, 2), :],
             ap2b_ref[:, pl.ds(di, 8, 2), pl.ds(dj, 8, 2), :]], axis=-1)
        acc = acc + jnp.dot(v.reshape(8 * 8 * 8, 256), w3_ref[s],
                            preferred_element_type=jnp.float32)
    a3 = jnp.maximum(acc + b3_ref[...], 0.0)

    z1 = jnp.mean(a3.reshape(8, 64, 256), axis=1)
    o_ref[...] = jnp.dot(z1, fcw_ref[...],
                         preferred_element_type=jnp.float32) + fcb_ref[...]


def _mamba_kernel(h_ref, nw_ref, ipw_ref, cw_ref, cb_ref, xpw_ref,
                  dpw_ref, dpb_ref, alt_ref, dp_ref, opw_ref, nfw_ref,
                  h1_ref, h1b_ref, h2_ref, h2b_ref, o_ref, s1_ref, s2_ref):
    h = h_ref[0]

    def rms(x):
        return x * jax.lax.rsqrt(
            jnp.mean(x * x, axis=-1, keepdims=True) + 1e-5)

    for l in range(_NL):
        xn = rms(h) * nw_ref[l]
        xz = jnp.dot(xn, ipw_ref[l], preferred_element_type=jnp.float32)
        xin, z = xz[:, :_E], xz[:, _E:]
        acc = xin * cw_ref[l, 3]
        for k in range(3):
            sh = 3 - k
            xsh = jnp.pad(xin, ((sh, 0), (0, 0)))[:_T]
            acc = acc + xsh * cw_ref[l, k]
        xc = jax.nn.silu(acc + cb_ref[l])
        dbc = jnp.dot(xc, xpw_ref[l], preferred_element_type=jnp.float32)
        dt, Bm, Cm = dbc[:, :_DTR], dbc[:, _DTR:_DTR + _N], dbc[:, _DTR + _N:]
        delta = jax.nn.softplus(
            jnp.dot(dt, dpw_ref[l], preferred_element_type=jnp.float32)
            + dpb_ref[l])
        A = -jnp.exp(alt_ref[l])
        s1_ref[...] = jnp.exp(delta[:, None, :] * A[None])
        s2_ref[...] = (delta * xc)[:, None, :] * Bm[:, :, None]

        def step(t, hst):
            hst = s1_ref[pl.ds(t, 1)] * hst + s2_ref[pl.ds(t, 1)]
            s1_ref[pl.ds(t, 1)] = hst
            return hst

        jax.lax.fori_loop(0, _T, step, jnp.zeros((1, _N, _E), jnp.float32))
        y = jnp.sum(s1_ref[...] * Cm[:, :, None], axis=1)
        y = y + xc * dp_ref[l]
        y = y * jax.nn.silu(z)
        h = h + jnp.dot(y, opw_ref[l], preferred_element_type=jnp.float32)

    z2 = rms(h) * nfw_ref[...]
    hh = jax.nn.gelu(jnp.dot(z2, h1_ref[...],
                             preferred_element_type=jnp.float32)
                     + h1b_ref[...])
    logits = jnp.sum(hh * h2_ref[...], axis=-1, keepdims=True) + h2b_ref[...]
    lt = logits.reshape(1, _T)
    m = jnp.max(lt, axis=-1, keepdims=True)
    e = jnp.exp(lt - m)
    w = e / jnp.sum(e, axis=-1, keepdims=True)
    idx = jax.lax.broadcasted_iota(jnp.int32, (1, _T), 1)
    o_ref[...] = jnp.where(idx == 0, 0.0, w)[None]


def kernel(x, cnn_w1, cnn_b1, cnn_w2, cnn_b2, cnn_w3, cnn_b3, fc_w, fc_b,
           norm_w, in_proj_w, conv1d_w, conv1d_b, x_proj_w, dt_proj_w,
           dt_proj_b, A_log, Dp, out_proj_w, norm_f_w, head_w1, head_b1,
           head_w2, head_b2):
    xf = x.reshape(_F, _H, _W)
    xp = jnp.pad(xf, ((0, 0), (0, 1), (0, 1)))
    patches1 = jnp.stack(
        [xp[:, di:di + 63:2, dj:dj + 63:2] for di, dj in _TAPS], axis=-1)
    p1i = patches1.reshape(_F // 8, 8, 32, 32, 9).transpose(
        0, 2, 3, 1, 4).reshape(_F // 8, 32, 32, 72)
    eye8 = jnp.eye(8, dtype=jnp.float32)
    w1m = cnn_w1.transpose(2, 3, 1, 0).reshape(9, 16)
    w2m = cnn_w2.transpose(2, 3, 1, 0).reshape(9, 16, 32)
    w3m = cnn_w3.transpose(2, 3, 1, 0).reshape(9, 32, 32)
    w1b = jnp.kron(eye8, w1m)
    w2b = jnp.stack([jnp.kron(eye8, w2m[s]) for s in range(9)])
    w3b = jnp.stack([jnp.kron(eye8, w3m[s]) for s in range(9)])
    fcb_ = jnp.kron(eye8, fc_w.T)

    h = pl.pallas_call(
        _cnn_kernel,
        grid=(_F // 64,),
        in_specs=[
            pl.BlockSpec((8, 32, 32, 72), lambda i: (i, 0, 0, 0)),
            pl.BlockSpec((72, 128), lambda i: (0, 0)),
            pl.BlockSpec((1, 128), lambda i: (0, 0)),
            pl.BlockSpec((9, 128, 256), lambda i: (0, 0, 0)),
            pl.BlockSpec((1, 256), lambda i: (0, 0)),
            pl.BlockSpec((9, 256, 256), lambda i: (0, 0, 0)),
            pl.BlockSpec((1, 256), lambda i: (0, 0)),
            pl.BlockSpec((256, 2048), lambda i: (0, 0)),
            pl.BlockSpec((1, 2048), lambda i: (0, 0)),
        ],
        out_specs=pl.BlockSpec((8, 2048), lambda i: (i, 0)),
        out_shape=jax.ShapeDtypeStruct((_F // 8, 2048), jnp.float32),
        scratch_shapes=[
            pltpu.VMEM((8, 34, 34, 128), jnp.float32),
            pltpu.VMEM((8, 18, 18, 128), jnp.float32),
            pltpu.VMEM((8, 18, 18, 128), jnp.float32),
        ],
        compiler_params=pltpu.CompilerParams(
            dimension_semantics=("parallel",),
            vmem_limit_bytes=56 * 1024 * 1024,
        ),
    )(p1i, w1b, jnp.tile(cnn_b1, 8)[None], w2b, jnp.tile(cnn_b2, 8)[None],
      w3b, jnp.tile(cnn_b3, 8)[None], fcb_, jnp.tile(fc_b, 8)[None])

    hseq = h.reshape(_F, _D).reshape(_B, _T, _D)

    w = pl.pallas_call(
        _mamba_kernel,
        grid=(_B,),
        in_specs=[
            pl.BlockSpec((1, _T, _D), lambda i: (i, 0, 0)),
            pl.BlockSpec((_NL, _D), lambda i: (0, 0)),
            pl.BlockSpec((_NL, _D, 2 * _E), lambda i: (0, 0, 0)),
            pl.BlockSpec((_NL, 4, _E), lambda i: (0, 0, 0)),
            pl.BlockSpec((_NL, _E), lambda i: (0, 0)),
            pl.BlockSpec((_NL, _E, _DTR + 2 * _N), lambda i: (0, 0, 0)),
            pl.BlockSpec((_NL, _DTR, _E), lambda i: (0, 0, 0)),
            pl.BlockSpec((_NL, _E), lambda i: (0, 0)),
            pl.BlockSpec((_NL, _N, _E), lambda i: (0, 0, 0)),
            pl.BlockSpec((_NL, _E), lambda i: (0, 0)),
            pl.BlockSpec((_NL, _E, _D), lambda i: (0, 0, 0)),
            pl.BlockSpec((1, _D), lambda i: (0, 0)),
            pl.BlockSpec((_D, 64), lambda i: (0, 0)),
            pl.BlockSpec((1, 64), lambda i: (0, 0)),
            pl.BlockSpec((1, 64), lambda i: (0, 0)),
            pl.BlockSpec((1, 1), lambda i: (0, 0)),
        ],
        out_specs=pl.BlockSpec((1, 1, _T), lambda i: (i, 0, 0)),
        out_shape=jax.ShapeDtypeStruct((_B, 1, _T), jnp.float32),
        scratch_shapes=[
            pltpu.VMEM((_T, _N, _E), jnp.float32),
            pltpu.VMEM((_T, _N, _E), jnp.float32),
        ],
        compiler_params=pltpu.CompilerParams(
            dimension_semantics=("parallel",),
            vmem_limit_bytes=56 * 1024 * 1024,
        ),
    )(hseq, norm_w, in_proj_w.transpose(0, 2, 1),
      conv1d_w.transpose(0, 2, 1), conv1d_b, x_proj_w.transpose(0, 2, 1),
      dt_proj_w.transpose(0, 2, 1), dt_proj_b, A_log.transpose(0, 2, 1),
      Dp, out_proj_w.transpose(0, 2, 1), norm_f_w.reshape(1, _D),
      head_w1.T, head_b1.reshape(1, 64), head_w2, head_b2.reshape(1, 1))

    return w.reshape(_B, _T, 1)
